# baseline (speedup 1.0000x reference)
import jax
import jax.numpy as jnp
from jax.experimental import pallas as pl

_BLOCK = 4000


def _ogc_matmul_kernel(x_ref, w_ref, o_ref):
    o_ref[...] = jax.lax.dot_general(
        x_ref[...],
        w_ref[...],
        dimension_numbers=(((1,), (1,)), ((), ())),
        preferred_element_type=jnp.float32,
    )


def kernel(x, W):
    n, nfeat = x.shape
    nclass = W.shape[0]
    grid = n // _BLOCK
    return pl.pallas_call(
        _ogc_matmul_kernel,
        grid=(grid,),
        in_specs=[
            pl.BlockSpec((_BLOCK, nfeat), lambda i: (i, 0)),
            pl.BlockSpec((nclass, nfeat), lambda i: (0, 0)),
        ],
        out_specs=pl.BlockSpec((_BLOCK, nclass), lambda i: (i, 0)),
        out_shape=jax.ShapeDtypeStruct((n, nclass), jnp.float32),
    )(x, W)


# speedup vs baseline: 1.0372x; 1.0372x over previous
import jax
import jax.numpy as jnp
from jax.experimental import pallas as pl
from jax.experimental.pallas import tpu as pltpu

_BLOCK = 2000
_DEPTH = 8


def _ogc_matmul_kernel(x_hbm, w_ref, o_ref, buf, sems):
    i = pl.program_id(0)
    nsteps = pl.num_programs(0)

    def tile_copy(step, slot):
        return pltpu.make_async_copy(
            x_hbm.at[pl.ds(step * _BLOCK, _BLOCK), :],
            buf.at[slot],
            sems.at[slot],
        )

    @pl.when(i == 0)
    def _warmup():
        for j in range(_DEPTH):
            tile_copy(j, j).start()

    slot = jax.lax.rem(i, _DEPTH)
    tile_copy(i, slot).wait()
    o_ref[...] = jax.lax.dot_general(
        buf[slot],
        w_ref[...],
        dimension_numbers=(((1,), (1,)), ((), ())),
        preferred_element_type=jnp.float32,
    )

    nxt = i + _DEPTH

    @pl.when(nxt < nsteps)
    def _prefetch():
        tile_copy(nxt, slot).start()


def kernel(x, W):
    n, nfeat = x.shape
    nclass = W.shape[0]
    grid = n // _BLOCK
    return pl.pallas_call(
        _ogc_matmul_kernel,
        grid=(grid,),
        in_specs=[
            pl.BlockSpec(memory_space=pl.ANY),
            pl.BlockSpec((nclass, nfeat), lambda i: (0, 0)),
        ],
        out_specs=pl.BlockSpec((_BLOCK, nclass), lambda i: (i, 0)),
        out_shape=jax.ShapeDtypeStruct((n, nclass), jnp.float32),
        scratch_shapes=[
            pltpu.VMEM((_DEPTH, _BLOCK, nfeat), jnp.float32),
            pltpu.SemaphoreType.DMA((_DEPTH,)),
        ],
        compiler_params=pltpu.CompilerParams(
            dimension_semantics=("arbitrary",),
        ),
    )(x, W)
